# baseline (speedup 1.0000x reference)
import functools

import jax
import jax.numpy as jnp
from jax import lax
from jax.experimental import pallas as pl
from jax.experimental.pallas import tpu as pltpu
from jax.experimental.pallas import tpu_sc as plsc

N_NODES = 10000
N_EDGES = 320000
DIM = 128
DOUT2 = 256

NC = 2
NS = 16
NW = NC * NS
N_PAD = 10240
ROWS_PER_TILE = N_PAD // NS
CNT_ROWS = N_PAD // DIM
ACC_ROWS = N_PAD + CNT_ROWS
EB = 64
N_BATCH = N_EDGES // EB
NBT = N_BATCH // NW
NGR = EB // 16

BLK = 1024


def _make_sc_segment_sum(with_count):
    mesh = plsc.VectorSubcoreMesh(core_axis_name="c", subcore_axis_name="s")
    nch = DIM // 16

    scratch = [
        pltpu.VMEM((EB,), jnp.int32),
        pltpu.VMEM((EB,), jnp.int32),
        pltpu.VMEM((EB,), jnp.int32),
        pltpu.VMEM((EB,), jnp.int32),
        pltpu.VMEM((EB,), jnp.float32),
        pltpu.VMEM((EB,), jnp.float32),
        pltpu.VMEM((EB,), jnp.int32),
        pltpu.VMEM((EB,), jnp.int32),
        pltpu.VMEM((EB,), jnp.int32),
        pltpu.VMEM((EB,), jnp.int32),
        pltpu.VMEM((EB,), jnp.float32),
        pltpu.VMEM((EB,), jnp.float32),
        pltpu.VMEM((EB, DIM), jnp.float32),
        pltpu.VMEM((EB, DIM), jnp.float32),
        pltpu.VMEM((EB, DIM), jnp.float32),
        pltpu.VMEM((EB, DIM), jnp.float32),
        pltpu.VMEM_SHARED((ACC_ROWS, DIM), jnp.float32),
        pltpu.SemaphoreType.DMA,
        pltpu.SemaphoreType.DMA,
        pltpu.SemaphoreType.DMA,
        pltpu.SemaphoreType.DMA,
        pltpu.SemaphoreType.DMA,
        pltpu.SemaphoreType.DMA,
    ]
    if with_count:
        scratch += [
            pltpu.VMEM((CNT_ROWS, DIM), jnp.float32),
            pltpu.VMEM((CNT_ROWS,), jnp.int32),
        ]

    @functools.partial(
        pl.kernel,
        out_type=jax.ShapeDtypeStruct((NC, ACC_ROWS, DIM), jnp.float32),
        mesh=mesh,
        scratch_types=scratch,
        compiler_params=pltpu.CompilerParams(needs_layout_passes=False),
    )
    def sc_kernel(x_hbm, src_hbm, dst_hbm, ew_hbm, out_hbm,
                  srcb0, srcb1, dstb0, dstb1, ewb0, ewb1,
                  srcg0, srcg1, dstg0, dstg1, ewg0, ewg1,
                  rows_g0, rows_g1, rows_s0, rows_s1, acc,
                  isem0, isem1, gsem0, gsem1, ssem0, ssem1,
                  cnt_v=None, cidx_v=None):
        c = lax.axis_index("c")
        s = lax.axis_index("s")
        w = c * NS + s

        srcb = (srcb0, srcb1)
        dstb = (dstb0, dstb1)
        ewb = (ewb0, ewb1)
        rows_g = (rows_g0, rows_g1)
        rows_s = (rows_s0, rows_s1)
        srcg = (srcg0, srcg1)
        dstg = (dstg0, dstg1)
        ewg = (ewg0, ewg1)
        isem = (isem0, isem1)
        gsem = (gsem0, gsem1)
        ssem = (ssem0, ssem1)

        zvec = jnp.zeros((16,), jnp.float32)
        ramp = lax.broadcasted_iota(jnp.int32, (16,), 0)
        ones16 = jnp.ones((16,), jnp.float32)

        def zero_row(r, carry):
            for j in range(nch):
                rows_s0[r, pl.ds(16 * j, 16)] = zvec
            return carry

        lax.fori_loop(0, EB, zero_row, 0)

        for k in range(ROWS_PER_TILE // EB):
            pltpu.sync_copy(
                rows_s0, acc.at[pl.ds(s * ROWS_PER_TILE + k * EB, EB)])
        nct = 8

        @pl.when(s < CNT_ROWS // nct)
        def _():
            pltpu.sync_copy(rows_s0.at[pl.ds(0, nct)],
                            acc.at[pl.ds(N_PAD + s * nct, nct)])

        if with_count:
            def zero_cnt(r, carry):
                for j in range(nch):
                    cnt_v[r, pl.ds(16 * j, 16)] = zvec
                return carry

            lax.fori_loop(0, CNT_ROWS, zero_cnt, 0)
            for k in range(CNT_ROWS // 16):
                cidx_v[pl.ds(16 * k, 16)] = ramp + (N_PAD + 16 * k)

        plsc.subcore_barrier()

        base0 = w * NBT

        def start_idx(bg, k):
            e0 = bg * EB
            pltpu.async_copy(src_hbm.at[pl.ds(e0, EB)], srcb[k], isem[k])
            pltpu.async_copy(dst_hbm.at[pl.ds(e0, EB)], dstb[k], isem[k])
            pltpu.async_copy(ew_hbm.at[pl.ds(e0, EB)], ewb[k], isem[k])

        def wait_idx(k):
            pltpu.make_async_copy(
                src_hbm.at[pl.ds(0, EB)], srcb[k], isem[k]).wait()
            pltpu.make_async_copy(
                dst_hbm.at[pl.ds(0, EB)], dstb[k], isem[k]).wait()
            pltpu.make_async_copy(
                ew_hbm.at[pl.ds(0, EB)], ewb[k], isem[k]).wait()

        def start_gather(k):
            pltpu.async_copy(x_hbm.at[srcg[k]], rows_g[k], gsem[k])

        def wait_gather(k):
            pltpu.make_async_copy(
                x_hbm.at[srcg[k]], rows_g[k], gsem[k]).wait()

        def stage_src(k):
            for g in range(NGR):
                sl = pl.ds(g * 16, 16)
                srcg[k][sl] = srcb[k][sl]

        def start_scatter(k):
            pltpu.async_copy(rows_s[k], acc.at[dstg[k]], ssem[k], add=True)

        def wait_scatter(k):
            pltpu.make_async_copy(rows_s[k], acc.at[dstg[k]], ssem[k]).wait()

        def stage_idx(k):
            for g in range(NGR):
                sl = pl.ds(g * 16, 16)
                dstg[k][sl] = dstb[k][sl]
                ewg[k][sl] = ewb[k][sl]

        def scale_batch(k):
            def group_body(g, inner):
                wchunk = ewg[k][pl.ds(g * 16, 16)]
                for r in range(16):
                    w16 = lax.gather(
                        wchunk,
                        jnp.full((16, 1), r, jnp.int32),
                        lax.GatherDimensionNumbers(
                            offset_dims=(),
                            collapsed_slice_dims=(0,),
                            start_index_map=(0,)),
                        slice_sizes=(1,),
                        mode=lax.GatherScatterMode.PROMISE_IN_BOUNDS)
                    row = g * 16 + r
                    for j in range(nch):
                        rows_s[k][row, pl.ds(16 * j, 16)] = (
                            rows_g[k][row, pl.ds(16 * j, 16)] * w16)
                if with_count:
                    dchunk = dstg[k][pl.ds(g * 16, 16)]
                    hi_i = lax.shift_right_logical(dchunk, 7)
                    lo_i = lax.bitwise_and(dchunk, 127)
                    plsc.addupdate_scatter(cnt_v, [hi_i, lo_i], ones16)
                return inner

            lax.fori_loop(0, NGR, group_body, 0)

        start_idx(base0, 0)
        start_idx(base0 + 1, 1)
        wait_idx(0)
        stage_src(0)
        start_gather(0)

        def pair_body(g2, carry):
            for k in range(2):
                bl = g2 * 2 + k
                wait_gather(k)

                @pl.when(bl >= 2)
                def _():
                    wait_scatter(k)

                stage_idx(k)

                @pl.when(bl + 2 < NBT)
                def _():
                    start_idx(base0 + bl + 2, k)

                @pl.when(bl + 1 < NBT)
                def _():
                    wait_idx(1 - k)
                    stage_src(1 - k)
                    start_gather(1 - k)

                scale_batch(k)
                start_scatter(k)
            return carry

        lax.fori_loop(0, NBT // 2, pair_body, 0)

        wait_scatter(0)
        wait_scatter(1)

        @pl.when(w < N_BATCH - NBT * NW)
        def _():
            start_idx(NBT * NW + w, 0)
            wait_idx(0)
            stage_src(0)
            pltpu.async_copy(x_hbm.at[srcg[0]], rows_g[0], gsem[0]).wait()
            stage_idx(0)
            scale_batch(0)
            pltpu.sync_copy(rows_s[0], acc.at[dstg[0]], add=True)

        if with_count:
            pltpu.sync_copy(cnt_v, acc.at[cidx_v], add=True)

        plsc.subcore_barrier()

        pltpu.sync_copy(
            acc.at[pl.ds(s * ROWS_PER_TILE, ROWS_PER_TILE)],
            out_hbm.at[c, pl.ds(s * ROWS_PER_TILE, ROWS_PER_TILE)])

        @pl.when(s < CNT_ROWS // nct)
        def _():
            pltpu.sync_copy(
                acc.at[pl.ds(N_PAD + s * nct, nct)],
                out_hbm.at[c, pl.ds(N_PAD + s * nct, nct)])

    return sc_kernel


_sc_pass1 = _make_sc_segment_sum(with_count=True)
_sc_pass2 = _make_sc_segment_sum(with_count=False)


def _tc_layer1(P, x, W_rel, b_rel, W_root):
    nblk = (N_NODES + BLK - 1) // BLK
    crpb = BLK // DIM

    def body(p_ref, c_ref, x_ref, wr_ref, b_ref, wo_ref, h_ref, inv_ref):
        ssum = p_ref[0] + p_ref[1]
        cnt8 = c_ref[0] + c_ref[1]
        n_i = lax.broadcasted_iota(jnp.int32, (BLK, crpb), 0)
        r_i = lax.broadcasted_iota(jnp.int32, (BLK, crpb), 1)
        rowsel = (lax.shift_right_logical(n_i, 7) == r_i).astype(jnp.float32)
        t1 = jnp.dot(rowsel, cnt8, preferred_element_type=jnp.float32)
        n2 = lax.broadcasted_iota(jnp.int32, (BLK, DIM), 0)
        c2 = lax.broadcasted_iota(jnp.int32, (BLK, DIM), 1)
        lanesel = (lax.bitwise_and(n2, 127) == c2).astype(jnp.float32)
        cnt = jnp.sum(t1 * lanesel, axis=1, keepdims=True)
        inv = 1.0 / (cnt + 1.0)
        xb = x_ref[...]
        agg = (ssum + xb) * inv
        h_ref[...] = (
            jnp.dot(agg, wr_ref[...], preferred_element_type=jnp.float32)
            + jnp.dot(xb, wo_ref[...], preferred_element_type=jnp.float32)
            + b_ref[...])
        inv_ref[...] = jnp.broadcast_to(inv, (BLK, DIM))

    return pl.pallas_call(
        body,
        grid=(nblk,),
        in_specs=[
            pl.BlockSpec((NC, BLK, DIM), lambda i: (0, i, 0)),
            pl.BlockSpec((NC, crpb, DIM), lambda i: (0, N_PAD // crpb + i, 0)),
            pl.BlockSpec((BLK, DIM), lambda i: (i, 0)),
            pl.BlockSpec((DIM, DIM), lambda i: (0, 0)),
            pl.BlockSpec((1, DIM), lambda i: (0, 0)),
            pl.BlockSpec((DIM, DIM), lambda i: (0, 0)),
        ],
        out_specs=[
            pl.BlockSpec((BLK, DIM), lambda i: (i, 0)),
            pl.BlockSpec((BLK, DIM), lambda i: (i, 0)),
        ],
        out_shape=[
            jax.ShapeDtypeStruct((N_NODES, DIM), jnp.float32),
            jax.ShapeDtypeStruct((N_NODES, DIM), jnp.float32),
        ],
    )(P, P, x, W_rel, b_rel.reshape(1, DIM), W_root)


def _tc_layer2(P2, h, inv, W_rel, b_rel, W_root):
    nblk = (N_NODES + BLK - 1) // BLK

    def body(p_ref, h_ref, inv_ref, wr_ref, b_ref, wo_ref, o_ref):
        s2 = p_ref[0] + p_ref[1]
        hb = h_ref[...]
        agg = (s2 + hb) * inv_ref[...]
        o_ref[...] = (
            jnp.dot(agg, wr_ref[...], preferred_element_type=jnp.float32)
            + jnp.dot(hb, wo_ref[...], preferred_element_type=jnp.float32)
            + b_ref[...])

    return pl.pallas_call(
        body,
        grid=(nblk,),
        in_specs=[
            pl.BlockSpec((NC, BLK, DIM), lambda i: (0, i, 0)),
            pl.BlockSpec((BLK, DIM), lambda i: (i, 0)),
            pl.BlockSpec((BLK, DIM), lambda i: (i, 0)),
            pl.BlockSpec((DIM, DOUT2), lambda i: (0, 0)),
            pl.BlockSpec((1, DOUT2), lambda i: (0, 0)),
            pl.BlockSpec((DIM, DOUT2), lambda i: (0, 0)),
        ],
        out_specs=pl.BlockSpec((BLK, DOUT2), lambda i: (i, 0)),
        out_shape=jax.ShapeDtypeStruct((N_NODES, DOUT2), jnp.float32),
    )(P2, h, inv, W_rel, b_rel.reshape(1, DOUT2), W_root)


def kernel(x, edge_index, edge_weight,
           W1_rel, b1_rel, W1_root, W2_rel, b2_rel, W2_root):
    src = edge_index[0].astype(jnp.int32)
    dst = edge_index[1].astype(jnp.int32)
    ew = edge_weight.astype(jnp.float32)

    P1 = _sc_pass1(x, src, dst, ew)
    h, inv = _tc_layer1(P1, x, W1_rel, b1_rel, W1_root)
    P2 = _sc_pass2(h, src, dst, ew)
    out = _tc_layer2(P2, h, inv, W2_rel, b2_rel, W2_root)
    return out


# speedup vs baseline: 1.2228x; 1.2228x over previous
import functools

import jax
import jax.numpy as jnp
from jax import lax
from jax.experimental import pallas as pl
from jax.experimental.pallas import tpu as pltpu
from jax.experimental.pallas import tpu_sc as plsc

N_NODES = 10000
N_EDGES = 320000
DIM = 128
DOUT2 = 256

NC = 2
NS = 16
NW = NC * NS
N_PAD = 10240
ROWS_PER_TILE = N_PAD // NS
CNT_ROWS = N_PAD // DIM
ACC_ROWS = N_PAD + CNT_ROWS

BLK = 1024


def _make_sc_segment_sum(with_count):
    mesh = plsc.VectorSubcoreMesh(core_axis_name="c", subcore_axis_name="s")
    nch = DIM // 16
    EB = 64 if with_count else 80
    N_BATCH = N_EDGES // EB
    NBT = N_BATCH // NW
    NGR = EB // 16
    LEFTOVER = N_BATCH - NBT * NW

    scratch = [
        pltpu.VMEM((EB,), jnp.int32),
        pltpu.VMEM((EB,), jnp.int32),
        pltpu.VMEM((EB,), jnp.int32),
        pltpu.VMEM((EB,), jnp.int32),
        pltpu.VMEM((EB,), jnp.float32),
        pltpu.VMEM((EB,), jnp.float32),
        pltpu.VMEM((EB,), jnp.int32),
        pltpu.VMEM((EB,), jnp.int32),
        pltpu.VMEM((EB,), jnp.float32),
        pltpu.VMEM((EB,), jnp.float32),
        pltpu.VMEM((EB, DIM), jnp.float32),
        pltpu.VMEM((EB, DIM), jnp.float32),
        pltpu.VMEM((EB, DIM), jnp.float32),
        pltpu.VMEM((EB, DIM), jnp.float32),
        pltpu.VMEM_SHARED((ACC_ROWS, DIM), jnp.float32),
        pltpu.SemaphoreType.DMA,
        pltpu.SemaphoreType.DMA,
        pltpu.SemaphoreType.DMA,
        pltpu.SemaphoreType.DMA,
        pltpu.SemaphoreType.DMA,
        pltpu.SemaphoreType.DMA,
    ]
    if with_count:
        scratch += [
            pltpu.VMEM((CNT_ROWS, DIM), jnp.float32),
            pltpu.VMEM((CNT_ROWS,), jnp.int32),
        ]

    @functools.partial(
        pl.kernel,
        out_type=jax.ShapeDtypeStruct((NC, ACC_ROWS, DIM), jnp.float32),
        mesh=mesh,
        scratch_types=scratch,
        compiler_params=pltpu.CompilerParams(needs_layout_passes=False),
    )
    def sc_kernel(x_hbm, src_hbm, dst_hbm, ew_hbm, out_hbm,
                  srcb0, srcb1, dstb0, dstb1, ewb0, ewb1,
                  dstg0, dstg1, ewg0, ewg1,
                  rows_g0, rows_g1, rows_s0, rows_s1, acc,
                  isem0, isem1, gsem0, gsem1, ssem0, ssem1,
                  cnt_v=None, cidx_v=None):
        c = lax.axis_index("c")
        s = lax.axis_index("s")
        w = c * NS + s

        srcb = (srcb0, srcb1)
        dstb = (dstb0, dstb1)
        ewb = (ewb0, ewb1)
        rows_g = (rows_g0, rows_g1)
        rows_s = (rows_s0, rows_s1)
        dstg = (dstg0, dstg1)
        ewg = (ewg0, ewg1)
        isem = (isem0, isem1)
        gsem = (gsem0, gsem1)
        ssem = (ssem0, ssem1)

        zvec = jnp.zeros((16,), jnp.float32)
        ramp = lax.broadcasted_iota(jnp.int32, (16,), 0)
        ones16 = jnp.ones((16,), jnp.float32)

        def zero_row(r, carry):
            for j in range(nch):
                rows_s0[r, pl.ds(16 * j, 16)] = zvec
            return carry

        lax.fori_loop(0, EB, zero_row, 0)

        for k in range(ROWS_PER_TILE // EB):
            pltpu.sync_copy(
                rows_s0, acc.at[pl.ds(s * ROWS_PER_TILE + k * EB, EB)])
        nct = 8

        @pl.when(s < CNT_ROWS // nct)
        def _():
            pltpu.sync_copy(rows_s0.at[pl.ds(0, nct)],
                            acc.at[pl.ds(N_PAD + s * nct, nct)])

        if with_count:
            def zero_cnt(r, carry):
                for j in range(nch):
                    cnt_v[r, pl.ds(16 * j, 16)] = zvec
                return carry

            lax.fori_loop(0, CNT_ROWS, zero_cnt, 0)
            for k in range(CNT_ROWS // 16):
                cidx_v[pl.ds(16 * k, 16)] = ramp + (N_PAD + 16 * k)

        plsc.subcore_barrier()

        base0 = w * NBT

        def start_idx(bg, k):
            e0 = bg * EB
            pltpu.async_copy(src_hbm.at[pl.ds(e0, EB)], srcb[k], isem[k])
            pltpu.async_copy(dst_hbm.at[pl.ds(e0, EB)], dstb[k], isem[k])
            pltpu.async_copy(ew_hbm.at[pl.ds(e0, EB)], ewb[k], isem[k])

        def wait_idx(k):
            pltpu.make_async_copy(
                src_hbm.at[pl.ds(0, EB)], srcb[k], isem[k]).wait()
            pltpu.make_async_copy(
                dst_hbm.at[pl.ds(0, EB)], dstb[k], isem[k]).wait()
            pltpu.make_async_copy(
                ew_hbm.at[pl.ds(0, EB)], ewb[k], isem[k]).wait()

        def start_gather(k):
            pltpu.async_copy(x_hbm.at[srcb[k]], rows_g[k], gsem[k])

        def wait_gather(k):
            pltpu.make_async_copy(
                x_hbm.at[srcb[k]], rows_g[k], gsem[k]).wait()

        def start_scatter(k):
            pltpu.async_copy(rows_s[k], acc.at[dstg[k]], ssem[k], add=True)

        def wait_scatter(k):
            pltpu.make_async_copy(rows_s[k], acc.at[dstg[k]], ssem[k]).wait()

        def stage_idx(k):
            for g in range(NGR):
                sl = pl.ds(g * 16, 16)
                dstg[k][sl] = dstb[k][sl]
                ewg[k][sl] = ewb[k][sl]

        def scale_batch(k):
            def group_body(g, inner):
                wchunk = ewg[k][pl.ds(g * 16, 16)]
                for r in range(16):
                    w16 = lax.gather(
                        wchunk,
                        jnp.full((16, 1), r, jnp.int32),
                        lax.GatherDimensionNumbers(
                            offset_dims=(),
                            collapsed_slice_dims=(0,),
                            start_index_map=(0,)),
                        slice_sizes=(1,),
                        mode=lax.GatherScatterMode.PROMISE_IN_BOUNDS)
                    row = g * 16 + r
                    for j in range(nch):
                        rows_s[k][row, pl.ds(16 * j, 16)] = (
                            rows_g[k][row, pl.ds(16 * j, 16)] * w16)
                if with_count:
                    dchunk = dstg[k][pl.ds(g * 16, 16)]
                    hi_i = lax.shift_right_logical(dchunk, 7)
                    lo_i = lax.bitwise_and(dchunk, 127)
                    plsc.addupdate_scatter(cnt_v, [hi_i, lo_i], ones16)
                return inner

            lax.fori_loop(0, NGR, group_body, 0)

        start_idx(base0, 0)
        start_idx(base0 + 1, 1)
        wait_idx(0)
        start_gather(0)
        wait_idx(1)
        start_gather(1)

        def slot(bl, k):
            wait_gather(k)

            @pl.when(bl >= 2)
            def _():
                wait_scatter(k)

            stage_idx(k)

            @pl.when(bl + 2 < NBT)
            def _():
                start_idx(base0 + bl + 2, k)

            scale_batch(k)
            start_scatter(k)

            @pl.when(bl + 2 < NBT)
            def _():
                wait_idx(k)
                start_gather(k)

        def pair_body(g2, carry):
            for k in range(2):
                slot(g2 * 2 + k, k)
            return carry

        lax.fori_loop(0, NBT // 2, pair_body, 0)
        if NBT % 2:
            slot(jnp.int32(NBT - 1), 0)

        wait_scatter(0)
        wait_scatter(1)

        if LEFTOVER:
            @pl.when(w < LEFTOVER)
            def _():
                start_idx(NBT * NW + w, 0)
                wait_idx(0)
                pltpu.async_copy(
                    x_hbm.at[srcb[0]], rows_g[0], gsem[0]).wait()
                stage_idx(0)
                scale_batch(0)
                pltpu.sync_copy(rows_s[0], acc.at[dstg[0]], add=True)

        if with_count:
            pltpu.sync_copy(cnt_v, acc.at[cidx_v], add=True)

        plsc.subcore_barrier()

        pltpu.sync_copy(
            acc.at[pl.ds(s * ROWS_PER_TILE, ROWS_PER_TILE)],
            out_hbm.at[c, pl.ds(s * ROWS_PER_TILE, ROWS_PER_TILE)])

        @pl.when(s < CNT_ROWS // nct)
        def _():
            pltpu.sync_copy(
                acc.at[pl.ds(N_PAD + s * nct, nct)],
                out_hbm.at[c, pl.ds(N_PAD + s * nct, nct)])

    return sc_kernel


_sc_pass1 = _make_sc_segment_sum(with_count=True)
_sc_pass2 = _make_sc_segment_sum(with_count=False)


def _tc_layer1(P, x, W_rel, b_rel, W_root):
    nblk = (N_NODES + BLK - 1) // BLK
    crpb = BLK // DIM

    def body(p_ref, c_ref, x_ref, wr_ref, b_ref, wo_ref, h_ref, inv_ref):
        ssum = p_ref[0] + p_ref[1]
        cnt8 = c_ref[0] + c_ref[1]
        n_i = lax.broadcasted_iota(jnp.int32, (BLK, crpb), 0)
        r_i = lax.broadcasted_iota(jnp.int32, (BLK, crpb), 1)
        rowsel = (lax.shift_right_logical(n_i, 7) == r_i).astype(jnp.float32)
        t1 = jnp.dot(rowsel, cnt8, preferred_element_type=jnp.float32)
        n2 = lax.broadcasted_iota(jnp.int32, (BLK, DIM), 0)
        c2 = lax.broadcasted_iota(jnp.int32, (BLK, DIM), 1)
        lanesel = (lax.bitwise_and(n2, 127) == c2).astype(jnp.float32)
        cnt = jnp.sum(t1 * lanesel, axis=1, keepdims=True)
        inv = 1.0 / (cnt + 1.0)
        xb = x_ref[...]
        agg = (ssum + xb) * inv
        h_ref[...] = (
            jnp.dot(agg, wr_ref[...], preferred_element_type=jnp.float32)
            + jnp.dot(xb, wo_ref[...], preferred_element_type=jnp.float32)
            + b_ref[...])
        inv_ref[...] = jnp.broadcast_to(inv, (BLK, DIM))

    return pl.pallas_call(
        body,
        grid=(nblk,),
        in_specs=[
            pl.BlockSpec((NC, BLK, DIM), lambda i: (0, i, 0)),
            pl.BlockSpec((NC, crpb, DIM), lambda i: (0, N_PAD // crpb + i, 0)),
            pl.BlockSpec((BLK, DIM), lambda i: (i, 0)),
            pl.BlockSpec((DIM, DIM), lambda i: (0, 0)),
            pl.BlockSpec((1, DIM), lambda i: (0, 0)),
            pl.BlockSpec((DIM, DIM), lambda i: (0, 0)),
        ],
        out_specs=[
            pl.BlockSpec((BLK, DIM), lambda i: (i, 0)),
            pl.BlockSpec((BLK, DIM), lambda i: (i, 0)),
        ],
        out_shape=[
            jax.ShapeDtypeStruct((N_NODES, DIM), jnp.float32),
            jax.ShapeDtypeStruct((N_NODES, DIM), jnp.float32),
        ],
    )(P, P, x, W_rel, b_rel.reshape(1, DIM), W_root)


def _tc_layer2(P2, h, inv, W_rel, b_rel, W_root):
    nblk = (N_NODES + BLK - 1) // BLK

    def body(p_ref, h_ref, inv_ref, wr_ref, b_ref, wo_ref, o_ref):
        s2 = p_ref[0] + p_ref[1]
        hb = h_ref[...]
        agg = (s2 + hb) * inv_ref[...]
        o_ref[...] = (
            jnp.dot(agg, wr_ref[...], preferred_element_type=jnp.float32)
            + jnp.dot(hb, wo_ref[...], preferred_element_type=jnp.float32)
            + b_ref[...])

    return pl.pallas_call(
        body,
        grid=(nblk,),
        in_specs=[
            pl.BlockSpec((NC, BLK, DIM), lambda i: (0, i, 0)),
            pl.BlockSpec((BLK, DIM), lambda i: (i, 0)),
            pl.BlockSpec((BLK, DIM), lambda i: (i, 0)),
            pl.BlockSpec((DIM, DOUT2), lambda i: (0, 0)),
            pl.BlockSpec((1, DOUT2), lambda i: (0, 0)),
            pl.BlockSpec((DIM, DOUT2), lambda i: (0, 0)),
        ],
        out_specs=pl.BlockSpec((BLK, DOUT2), lambda i: (i, 0)),
        out_shape=jax.ShapeDtypeStruct((N_NODES, DOUT2), jnp.float32),
    )(P2, h, inv, W_rel, b_rel.reshape(1, DOUT2), W_root)


def kernel(x, edge_index, edge_weight,
           W1_rel, b1_rel, W1_root, W2_rel, b2_rel, W2_root):
    src = edge_index[0].astype(jnp.int32)
    dst = edge_index[1].astype(jnp.int32)
    ew = edge_weight.astype(jnp.float32)

    P1 = _sc_pass1(x, src, dst, ew)
    h, inv = _tc_layer1(P1, x, W1_rel, b1_rel, W1_root)
    P2 = _sc_pass2(h, src, dst, ew)
    out = _tc_layer2(P2, h, inv, W2_rel, b2_rel, W2_root)
    return out


# speedup vs baseline: 1.2309x; 1.0066x over previous
import functools

import jax
import jax.numpy as jnp
from jax import lax
from jax.experimental import pallas as pl
from jax.experimental.pallas import tpu as pltpu
from jax.experimental.pallas import tpu_sc as plsc

N_NODES = 10000
N_EDGES = 320000
DIM = 128
DOUT2 = 256

NC = 2
NS = 16
NW = NC * NS
N_PAD = 10240
ROWS_PER_TILE = N_PAD // NS
CNT_ROWS = N_PAD // DIM
ACC_ROWS = N_PAD + CNT_ROWS

BLK = 1024


def _make_sc_segment_sum(with_count):
    mesh = plsc.VectorSubcoreMesh(core_axis_name="c", subcore_axis_name="s")
    nch = DIM // 16
    EB = 64 if with_count else 80
    N_BATCH = N_EDGES // EB
    NBT = N_BATCH // NW
    NGR = EB // 16
    LEFTOVER = N_BATCH - NBT * NW

    scratch = [
        pltpu.VMEM((EB,), jnp.int32),
        pltpu.VMEM((EB,), jnp.int32),
        pltpu.VMEM((EB,), jnp.int32),
        pltpu.VMEM((EB,), jnp.int32),
        pltpu.VMEM((EB,), jnp.float32),
        pltpu.VMEM((EB,), jnp.float32),
        pltpu.VMEM((EB,), jnp.int32),
        pltpu.VMEM((EB,), jnp.int32),
        pltpu.VMEM((EB,), jnp.float32),
        pltpu.VMEM((EB,), jnp.float32),
        pltpu.VMEM((EB, DIM), jnp.float32),
        pltpu.VMEM((EB, DIM), jnp.float32),
        pltpu.VMEM((EB, DIM), jnp.float32),
        pltpu.VMEM((EB, DIM), jnp.float32),
        pltpu.VMEM_SHARED((ACC_ROWS, DIM), jnp.float32),
        pltpu.SemaphoreType.DMA,
        pltpu.SemaphoreType.DMA,
        pltpu.SemaphoreType.DMA,
        pltpu.SemaphoreType.DMA,
        pltpu.SemaphoreType.DMA,
        pltpu.SemaphoreType.DMA,
    ]
    if with_count:
        scratch += [
            pltpu.VMEM((CNT_ROWS, DIM), jnp.float32),
            pltpu.VMEM((CNT_ROWS,), jnp.int32),
        ]

    @functools.partial(
        pl.kernel,
        out_type=jax.ShapeDtypeStruct((NC, ACC_ROWS, DIM), jnp.float32),
        mesh=mesh,
        scratch_types=scratch,
        compiler_params=pltpu.CompilerParams(needs_layout_passes=False),
    )
    def sc_kernel(x_hbm, src_hbm, dst_hbm, ew_hbm, out_hbm,
                  srcb0, srcb1, dstb0, dstb1, ewb0, ewb1,
                  dstg0, dstg1, ewg0, ewg1,
                  rows_g0, rows_g1, rows_s0, rows_s1, acc,
                  isem0, isem1, gsem0, gsem1, ssem0, ssem1,
                  cnt_v=None, cidx_v=None):
        c = lax.axis_index("c")
        s = lax.axis_index("s")
        w = c * NS + s

        srcb = (srcb0, srcb1)
        dstb = (dstb0, dstb1)
        ewb = (ewb0, ewb1)
        rows_g = (rows_g0, rows_g1)
        rows_s = (rows_s0, rows_s1)
        dstg = (dstg0, dstg1)
        ewg = (ewg0, ewg1)
        isem = (isem0, isem1)
        gsem = (gsem0, gsem1)
        ssem = (ssem0, ssem1)

        zvec = jnp.zeros((16,), jnp.float32)
        ramp = lax.broadcasted_iota(jnp.int32, (16,), 0)
        ones16 = jnp.ones((16,), jnp.float32)

        base0 = w * NBT

        def start_idx(bg, k):
            e0 = bg * EB
            pltpu.async_copy(src_hbm.at[pl.ds(e0, EB)], srcb[k], isem[k])
            pltpu.async_copy(dst_hbm.at[pl.ds(e0, EB)], dstb[k], isem[k])
            pltpu.async_copy(ew_hbm.at[pl.ds(e0, EB)], ewb[k], isem[k])

        def wait_idx(k):
            pltpu.make_async_copy(
                src_hbm.at[pl.ds(0, EB)], srcb[k], isem[k]).wait()
            pltpu.make_async_copy(
                dst_hbm.at[pl.ds(0, EB)], dstb[k], isem[k]).wait()
            pltpu.make_async_copy(
                ew_hbm.at[pl.ds(0, EB)], ewb[k], isem[k]).wait()

        def start_gather(k):
            pltpu.async_copy(x_hbm.at[srcb[k]], rows_g[k], gsem[k])

        def wait_gather(k):
            pltpu.make_async_copy(
                x_hbm.at[srcb[k]], rows_g[k], gsem[k]).wait()

        def start_scatter(k):
            pltpu.async_copy(rows_s[k], acc.at[dstg[k]], ssem[k], add=True)

        def wait_scatter(k):
            pltpu.make_async_copy(rows_s[k], acc.at[dstg[k]], ssem[k]).wait()

        def stage_idx(k):
            for g in range(NGR):
                sl = pl.ds(g * 16, 16)
                dstg[k][sl] = dstb[k][sl]
                ewg[k][sl] = ewb[k][sl]

        def scale_batch(k):
            def group_body(g, inner):
                wchunk = ewg[k][pl.ds(g * 16, 16)]
                for r in range(16):
                    w16 = lax.gather(
                        wchunk,
                        jnp.full((16, 1), r, jnp.int32),
                        lax.GatherDimensionNumbers(
                            offset_dims=(),
                            collapsed_slice_dims=(0,),
                            start_index_map=(0,)),
                        slice_sizes=(1,),
                        mode=lax.GatherScatterMode.PROMISE_IN_BOUNDS)
                    row = g * 16 + r
                    for j in range(nch):
                        rows_s[k][row, pl.ds(16 * j, 16)] = (
                            rows_g[k][row, pl.ds(16 * j, 16)] * w16)
                if with_count:
                    dchunk = dstg[k][pl.ds(g * 16, 16)]
                    hi_i = lax.shift_right_logical(dchunk, 7)
                    lo_i = lax.bitwise_and(dchunk, 127)
                    plsc.addupdate_scatter(cnt_v, [hi_i, lo_i], ones16)
                return inner

            lax.fori_loop(0, NGR, group_body, 0)

        start_idx(base0, 0)
        start_idx(base0 + 1, 1)
        wait_idx(0)
        start_gather(0)
        wait_idx(1)
        start_gather(1)

        def zero_row(r, carry):
            for j in range(nch):
                rows_s0[r, pl.ds(16 * j, 16)] = zvec
            return carry

        lax.fori_loop(0, EB, zero_row, 0)

        for k in range(ROWS_PER_TILE // EB):
            pltpu.sync_copy(
                rows_s0, acc.at[pl.ds(s * ROWS_PER_TILE + k * EB, EB)])
        nct = 8

        @pl.when(s < CNT_ROWS // nct)
        def _():
            pltpu.sync_copy(rows_s0.at[pl.ds(0, nct)],
                            acc.at[pl.ds(N_PAD + s * nct, nct)])

        if with_count:
            def zero_cnt(r, carry):
                for j in range(nch):
                    cnt_v[r, pl.ds(16 * j, 16)] = zvec
                return carry

            lax.fori_loop(0, CNT_ROWS, zero_cnt, 0)
            for k in range(CNT_ROWS // 16):
                cidx_v[pl.ds(16 * k, 16)] = ramp + (N_PAD + 16 * k)

        plsc.subcore_barrier()


        def slot(bl, k):
            wait_gather(k)

            @pl.when(bl >= 2)
            def _():
                wait_scatter(k)

            stage_idx(k)

            @pl.when(bl + 2 < NBT)
            def _():
                start_idx(base0 + bl + 2, k)

            scale_batch(k)
            start_scatter(k)

            @pl.when(bl + 2 < NBT)
            def _():
                wait_idx(k)
                start_gather(k)

        def pair_body(g2, carry):
            for k in range(2):
                slot(g2 * 2 + k, k)
            return carry

        lax.fori_loop(0, NBT // 2, pair_body, 0)
        if NBT % 2:
            slot(jnp.int32(NBT - 1), 0)

        wait_scatter(0)
        wait_scatter(1)

        if LEFTOVER:
            @pl.when(w < LEFTOVER)
            def _():
                start_idx(NBT * NW + w, 0)
                wait_idx(0)
                pltpu.async_copy(
                    x_hbm.at[srcb[0]], rows_g[0], gsem[0]).wait()
                stage_idx(0)
                scale_batch(0)
                pltpu.sync_copy(rows_s[0], acc.at[dstg[0]], add=True)

        if with_count:
            pltpu.sync_copy(cnt_v, acc.at[cidx_v], add=True)

        plsc.subcore_barrier()

        pltpu.sync_copy(
            acc.at[pl.ds(s * ROWS_PER_TILE, ROWS_PER_TILE)],
            out_hbm.at[c, pl.ds(s * ROWS_PER_TILE, ROWS_PER_TILE)])

        @pl.when(s < CNT_ROWS // nct)
        def _():
            pltpu.sync_copy(
                acc.at[pl.ds(N_PAD + s * nct, nct)],
                out_hbm.at[c, pl.ds(N_PAD + s * nct, nct)])

    return sc_kernel


_sc_pass1 = _make_sc_segment_sum(with_count=True)
_sc_pass2 = _make_sc_segment_sum(with_count=False)


def _tc_layer1(P, x, W_rel, b_rel, W_root):
    nblk = (N_NODES + BLK - 1) // BLK
    crpb = BLK // DIM

    def body(p_ref, c_ref, x_ref, wr_ref, b_ref, wo_ref, h_ref, inv_ref):
        ssum = p_ref[0] + p_ref[1]
        cnt8 = c_ref[0] + c_ref[1]
        n_i = lax.broadcasted_iota(jnp.int32, (BLK, crpb), 0)
        r_i = lax.broadcasted_iota(jnp.int32, (BLK, crpb), 1)
        rowsel = (lax.shift_right_logical(n_i, 7) == r_i).astype(jnp.float32)
        t1 = jnp.dot(rowsel, cnt8, preferred_element_type=jnp.float32)
        n2 = lax.broadcasted_iota(jnp.int32, (BLK, DIM), 0)
        c2 = lax.broadcasted_iota(jnp.int32, (BLK, DIM), 1)
        lanesel = (lax.bitwise_and(n2, 127) == c2).astype(jnp.float32)
        cnt = jnp.sum(t1 * lanesel, axis=1, keepdims=True)
        inv = 1.0 / (cnt + 1.0)
        xb = x_ref[...]
        agg = (ssum + xb) * inv
        h_ref[...] = (
            jnp.dot(agg, wr_ref[...], preferred_element_type=jnp.float32)
            + jnp.dot(xb, wo_ref[...], preferred_element_type=jnp.float32)
            + b_ref[...])
        inv_ref[...] = jnp.broadcast_to(inv, (BLK, DIM))

    return pl.pallas_call(
        body,
        grid=(nblk,),
        in_specs=[
            pl.BlockSpec((NC, BLK, DIM), lambda i: (0, i, 0)),
            pl.BlockSpec((NC, crpb, DIM), lambda i: (0, N_PAD // crpb + i, 0)),
            pl.BlockSpec((BLK, DIM), lambda i: (i, 0)),
            pl.BlockSpec((DIM, DIM), lambda i: (0, 0)),
            pl.BlockSpec((1, DIM), lambda i: (0, 0)),
            pl.BlockSpec((DIM, DIM), lambda i: (0, 0)),
        ],
        out_specs=[
            pl.BlockSpec((BLK, DIM), lambda i: (i, 0)),
            pl.BlockSpec((BLK, DIM), lambda i: (i, 0)),
        ],
        out_shape=[
            jax.ShapeDtypeStruct((N_NODES, DIM), jnp.float32),
            jax.ShapeDtypeStruct((N_NODES, DIM), jnp.float32),
        ],
    )(P, P, x, W_rel, b_rel.reshape(1, DIM), W_root)


def _tc_layer2(P2, h, inv, W_rel, b_rel, W_root):
    nblk = (N_NODES + BLK - 1) // BLK

    def body(p_ref, h_ref, inv_ref, wr_ref, b_ref, wo_ref, o_ref):
        s2 = p_ref[0] + p_ref[1]
        hb = h_ref[...]
        agg = (s2 + hb) * inv_ref[...]
        o_ref[...] = (
            jnp.dot(agg, wr_ref[...], preferred_element_type=jnp.float32)
            + jnp.dot(hb, wo_ref[...], preferred_element_type=jnp.float32)
            + b_ref[...])

    return pl.pallas_call(
        body,
        grid=(nblk,),
        in_specs=[
            pl.BlockSpec((NC, BLK, DIM), lambda i: (0, i, 0)),
            pl.BlockSpec((BLK, DIM), lambda i: (i, 0)),
            pl.BlockSpec((BLK, DIM), lambda i: (i, 0)),
            pl.BlockSpec((DIM, DOUT2), lambda i: (0, 0)),
            pl.BlockSpec((1, DOUT2), lambda i: (0, 0)),
            pl.BlockSpec((DIM, DOUT2), lambda i: (0, 0)),
        ],
        out_specs=pl.BlockSpec((BLK, DOUT2), lambda i: (i, 0)),
        out_shape=jax.ShapeDtypeStruct((N_NODES, DOUT2), jnp.float32),
    )(P2, h, inv, W_rel, b_rel.reshape(1, DOUT2), W_root)


def kernel(x, edge_index, edge_weight,
           W1_rel, b1_rel, W1_root, W2_rel, b2_rel, W2_root):
    src = edge_index[0].astype(jnp.int32)
    dst = edge_index[1].astype(jnp.int32)
    ew = edge_weight.astype(jnp.float32)

    P1 = _sc_pass1(x, src, dst, ew)
    h, inv = _tc_layer1(P1, x, W1_rel, b1_rel, W1_root)
    P2 = _sc_pass2(h, src, dst, ew)
    out = _tc_layer2(P2, h, inv, W2_rel, b2_rel, W2_root)
    return out
